# baseline (speedup 1.0000x reference)
import functools

import jax
import jax.numpy as jnp
from jax import lax
from jax.experimental import pallas as pl
from jax.experimental.pallas import tpu as pltpu
from jax.experimental.pallas import tpu_sc as plsc


def kernel(x, class_ids, embedding_table):
    del x
    B = class_ids.shape[0]
    V, D = embedding_table.shape
    table_t = embedding_table.T

    info = plsc.get_sparse_core_info()
    NC, NS, L = info.num_cores, info.num_subcores, info.num_lanes
    NW = NC * NS
    BLK = 128
    SW = 5
    STRIPE = SW * BLK
    NB_ALL = -(-V // BLK)
    q, r = divmod(NB_ALL, NW)
    NSTRIPE = -(-(q + 1) // SW)
    OFF_MAX = (NB_ALL - SW) * BLK
    NCHUNK = B // L
    NBUF = 2
    GROUPS = 8
    ROWS = GROUPS * L

    mesh = plsc.VectorSubcoreMesh(core_axis_name="c", subcore_axis_name="s")

    @functools.partial(
        pl.kernel,
        mesh=mesh,
        out_type=jax.ShapeDtypeStruct((B + L, D), jnp.float32),
        scratch_types=[
            pltpu.VMEM((B,), jnp.int32),
            pltpu.VMEM((B + L,), jnp.int32),
            pltpu.VMEM((B + L,), jnp.int32),
            pltpu.VMEM((NBUF, SW, D, BLK), jnp.float32),
            pltpu.VMEM((ROWS, BLK), jnp.float32),
            pltpu.SemaphoreType.DMA,
            pltpu.SemaphoreType.DMA,
            pltpu.SemaphoreType.DMA,
            pltpu.SemaphoreType.DMA,
        ],
        compiler_params=pltpu.CompilerParams(needs_layout_passes=False),
    )
    def gather_kernel(idx_hbm, table_hbm, out_hbm,
                      idx_v, loc_pk, hit_pk, blk, rows,
                      sem_blk0, sem_blk1, sem_blk2, sem_row):
        wid = lax.axis_index("s") * NC + lax.axis_index("c")
        lanes = lax.iota(jnp.int32, L)

        nb = jnp.where(wid < r, q + 1, q).astype(jnp.int32)
        base = jnp.where(wid < r, (q + 1) * wid, q * wid + r).astype(jnp.int32)
        lo = base * BLK
        hi = jnp.minimum((base + nb) * BLK, V)

        def stripe_off(s):
            return pl.multiple_of(
                jnp.minimum((base + s * SW) * BLK, OFF_MAX), BLK)

        def fire_stripe(s, buf, sem):
            off = stripe_off(s)
            for bb in range(SW):
                for i8 in range(D // 8):
                    pltpu.make_async_copy(
                        table_hbm.at[pl.ds(8 * i8, 8),
                                     pl.ds(off + bb * BLK, BLK)],
                        blk.at[buf, bb, pl.ds(8 * i8, 8)], sem,
                    ).start()

        fire_stripe(jnp.int32(0), 0, sem_blk0)
        fire_stripe(jnp.int32(1), 1, sem_blk1)

        pltpu.sync_copy(idx_hbm, idx_v)

        def filt(j, cnt):
            for u in range(4):
                jj = j * 4 + u
                v = idx_v[pl.ds(jj * L, L)]
                m = (v >= lo) & (v < hi)
                pk = v * (B + L) + lanes + jj * L
                c = plsc.cumsum(m.astype(jnp.int32))
                plsc.store_scatter(loc_pk, [cnt + c - 1], pk, mask=m)
                cnt = cnt + c[L - 1]
            return cnt

        cnt = lax.fori_loop(0, NCHUNK // 4, filt, jnp.int32(0))
        nloc = (cnt + L - 1) // L

        def stripe_iter(s, carry):
            fired = carry
            buf = lax.rem(s, NBUF)

            def wait_stripe(bufc, sem):
                for bb in range(SW):
                    pltpu.make_async_copy(
                        table_hbm.at[:, pl.ds(0, BLK)], blk.at[bufc, bb], sem
                    ).wait()

            @pl.when(buf == 0)
            def _():
                wait_stripe(0, sem_blk0)

            @pl.when(buf == 1)
            def _():
                wait_stripe(1, sem_blk1)
            c0 = stripe_off(s)
            pkbase = c0 * (B + L)

            def sfilt(m2, sh):
                pkv = loc_pk[pl.ds(m2 * L, L)]
                v = pkv // (B + L)
                m = ((lanes + m2 * L) < cnt) & (v >= c0) & (v < c0 + STRIPE)
                c = plsc.cumsum(m.astype(jnp.int32))
                plsc.store_scatter(hit_pk, [sh + c - 1], pkv - pkbase, mask=m)
                return sh + c[L - 1]

            sh = lax.fori_loop(0, nloc, sfilt, jnp.int32(0))

            def dense(m3, fired_in):
                pkv = hit_pk[pl.ds(m3 * L, L)]
                valid = (lanes + m3 * L) < sh
                colv = jnp.where(valid, pkv // (B + L), jnp.int32(0))
                posv = jnp.where(valid, lax.rem(pkv, B + L), B + lanes)
                g = lax.rem(fired_in, GROUPS)
                gbase = g * L

                @pl.when((fired_in > 0) & (lax.rem(fired_in, GROUPS) == 0))
                def _():
                    def fdrain(_, cc):
                        pltpu.make_async_copy(
                            out_hbm.at[0], rows.at[0, pl.ds(0, D)], sem_row
                        ).wait()
                        return cc
                    lax.fori_loop(0, GROUPS * L, fdrain, jnp.int32(0))

                slotv = gbase + lanes
                bufv = jnp.full((L,), 0, jnp.int32) + buf
                cbv = colv // BLK
                clv = lax.rem(colv, BLK)
                for d in range(D):
                    vals = plsc.load_gather(
                        blk, [bufv, cbv, jnp.full((L,), d, jnp.int32), clv])
                    plsc.store_scatter(rows, [slotv, jnp.full((L,), d, jnp.int32)],
                                       vals)
                for k in range(L):
                    pltpu.make_async_copy(
                        rows.at[gbase + k, pl.ds(0, D)],
                        out_hbm.at[posv[k]], sem_row,
                    ).start()
                return fired_in + 1

            nden = (sh + L - 1) // L
            fired = lax.fori_loop(0, nden, dense, fired)

            @pl.when((s + NBUF < NSTRIPE) & (buf == 0))
            def _():
                fire_stripe(s + NBUF, 0, sem_blk0)

            @pl.when((s + NBUF < NSTRIPE) & (buf == 1))
            def _():
                fire_stripe(s + NBUF, 1, sem_blk1)
            return fired

        fired = lax.fori_loop(0, NSTRIPE, stripe_iter, jnp.int32(0))

        def drain(_, carry):
            pltpu.make_async_copy(
                out_hbm.at[0], rows.at[0, pl.ds(0, D)], sem_row
            ).wait()
            return carry

        ndrain = jnp.where(
            fired > 0, fired - ((fired - 1) // GROUPS) * GROUPS, 0) * L
        lax.fori_loop(0, ndrain, drain, jnp.int32(0))

    out_full = gather_kernel(class_ids.astype(jnp.int32), table_t)
    return out_full[:B]


# speedup vs baseline: 1.0774x; 1.0774x over previous
import functools

import jax
import jax.numpy as jnp
from jax import lax
from jax.experimental import pallas as pl
from jax.experimental.pallas import tpu as pltpu
from jax.experimental.pallas import tpu_sc as plsc


def kernel(x, class_ids, embedding_table):
    del x
    B = class_ids.shape[0]
    V, D = embedding_table.shape
    table_t = embedding_table.T

    info = plsc.get_sparse_core_info()
    NC, NS, L = info.num_cores, info.num_subcores, info.num_lanes
    NW = NC * NS
    BLK = 128
    SW = 5
    STRIPE = SW * BLK
    NB_ALL = -(-V // BLK)
    q, r = divmod(NB_ALL, NW)
    NSTRIPE = -(-(q + 1) // SW)
    OFF_MAX = (NB_ALL - SW) * BLK
    NCHUNK = B // L
    NBUF = 2
    GROUPS = 8
    ROWS = GROUPS * L

    mesh = plsc.VectorSubcoreMesh(core_axis_name="c", subcore_axis_name="s")

    @functools.partial(
        pl.kernel,
        mesh=mesh,
        out_type=jax.ShapeDtypeStruct((B + L, D), jnp.float32),
        scratch_types=[
            pltpu.VMEM((B,), jnp.int32),
            pltpu.VMEM((B + L,), jnp.int32),
            pltpu.VMEM((B + L,), jnp.int32),
            pltpu.VMEM((NBUF, D, STRIPE), jnp.float32),
            pltpu.VMEM((ROWS, BLK), jnp.float32),
            pltpu.SemaphoreType.DMA,
            pltpu.SemaphoreType.DMA,
            pltpu.SemaphoreType.DMA,
            pltpu.SemaphoreType.DMA,
        ],
        compiler_params=pltpu.CompilerParams(needs_layout_passes=False),
    )
    def gather_kernel(idx_hbm, table_hbm, out_hbm,
                      idx_v, loc_pk, hit_pk, blk, rows,
                      sem_blk0, sem_blk1, sem_blk2, sem_row):
        wid = lax.axis_index("s") * NC + lax.axis_index("c")
        lanes = lax.iota(jnp.int32, L)

        nb = jnp.where(wid < r, q + 1, q).astype(jnp.int32)
        base = jnp.where(wid < r, (q + 1) * wid, q * wid + r).astype(jnp.int32)
        lo = base * BLK
        hi = jnp.minimum((base + nb) * BLK, V)

        def stripe_off(s):
            return pl.multiple_of(
                jnp.minimum((base + s * SW) * BLK, OFF_MAX), BLK)

        def fire_stripe(s, buf, sem):
            pltpu.make_async_copy(
                table_hbm.at[:, pl.ds(stripe_off(s), STRIPE)],
                blk.at[buf], sem,
            ).start()

        fire_stripe(jnp.int32(0), 0, sem_blk0)
        fire_stripe(jnp.int32(1), 1, sem_blk1)

        pltpu.sync_copy(idx_hbm, idx_v)

        def filt(j, cnt):
            for u in range(4):
                jj = j * 4 + u
                v = idx_v[pl.ds(jj * L, L)]
                m = (v >= lo) & (v < hi)
                pk = v * (B + L) + lanes + jj * L
                c = plsc.cumsum(m.astype(jnp.int32))
                plsc.store_scatter(loc_pk, [cnt + c - 1], pk, mask=m)
                cnt = cnt + c[L - 1]
            return cnt

        cnt = lax.fori_loop(0, NCHUNK // 4, filt, jnp.int32(0))
        nloc = (cnt + L - 1) // L

        def stripe_iter(s, carry):
            fired = carry
            buf = lax.rem(s, NBUF)

            def wait_stripe(bufc, sem):
                pltpu.make_async_copy(
                    table_hbm.at[:, pl.ds(0, STRIPE)], blk.at[bufc], sem
                ).wait()

            @pl.when(buf == 0)
            def _():
                wait_stripe(0, sem_blk0)

            @pl.when(buf == 1)
            def _():
                wait_stripe(1, sem_blk1)
            c0 = stripe_off(s)
            pkbase = c0 * (B + L)

            def sfilt(m2, sh):
                pkv = loc_pk[pl.ds(m2 * L, L)]
                v = pkv // (B + L)
                m = ((lanes + m2 * L) < cnt) & (v >= c0) & (v < c0 + STRIPE)
                c = plsc.cumsum(m.astype(jnp.int32))
                plsc.store_scatter(hit_pk, [sh + c - 1], pkv - pkbase, mask=m)
                return sh + c[L - 1]

            sh = lax.fori_loop(0, nloc, sfilt, jnp.int32(0))

            def dense(m3, fired_in):
                pkv = hit_pk[pl.ds(m3 * L, L)]
                valid = (lanes + m3 * L) < sh
                colv = jnp.where(valid, pkv // (B + L), jnp.int32(0))
                posv = jnp.where(valid, lax.rem(pkv, B + L), B + lanes)
                g = lax.rem(fired_in, GROUPS)
                gbase = g * L

                @pl.when((fired_in > 0) & (lax.rem(fired_in, GROUPS) == 0))
                def _():
                    def fdrain(_, cc):
                        pltpu.make_async_copy(
                            out_hbm.at[0], rows.at[0, pl.ds(0, D)], sem_row
                        ).wait()
                        return cc
                    lax.fori_loop(0, GROUPS * L, fdrain, jnp.int32(0))

                slotv = gbase + lanes
                bufv = jnp.full((L,), 0, jnp.int32) + buf
                for d in range(D):
                    vals = plsc.load_gather(
                        blk, [bufv, jnp.full((L,), d, jnp.int32), colv])
                    plsc.store_scatter(rows, [slotv, jnp.full((L,), d, jnp.int32)],
                                       vals)
                for k in range(L):
                    pltpu.make_async_copy(
                        rows.at[gbase + k, pl.ds(0, D)],
                        out_hbm.at[posv[k]], sem_row,
                    ).start()
                return fired_in + 1

            nden = (sh + L - 1) // L
            fired = lax.fori_loop(0, nden, dense, fired)

            @pl.when((s + NBUF < NSTRIPE) & (buf == 0))
            def _():
                fire_stripe(s + NBUF, 0, sem_blk0)

            @pl.when((s + NBUF < NSTRIPE) & (buf == 1))
            def _():
                fire_stripe(s + NBUF, 1, sem_blk1)
            return fired

        fired = lax.fori_loop(0, NSTRIPE, stripe_iter, jnp.int32(0))

        def drain(_, carry):
            pltpu.make_async_copy(
                out_hbm.at[0], rows.at[0, pl.ds(0, D)], sem_row
            ).wait()
            return carry

        ndrain = jnp.where(
            fired > 0, fired - ((fired - 1) // GROUPS) * GROUPS, 0) * L
        lax.fori_loop(0, ndrain, drain, jnp.int32(0))

    out_full = gather_kernel(class_ids.astype(jnp.int32), table_t)
    return out_full[:B]
